# baseline (speedup 1.0000x reference)
import jax
import jax.numpy as jnp
from jax.experimental import pallas as pl

N = 10000
EMB = 128
SDIM = 16
HID = 32
L = 5
VOCAB = 100
G = 64

_HI = jax.lax.Precision.HIGHEST
_BF = jnp.bfloat16
_F32 = jnp.float32


def _dot_bf(a, b):
    return jnp.dot(a.astype(_BF), b.astype(_BF), preferred_element_type=_F32)


def _hmp_kernel(atoms_ref, batch_ref, emb_ref, emb16T_ref, w1T_ref, b1_ref,
                w2T_ref, b2_ref, pw1_ref, pb1_ref, pw2_ref, pb2_ref,
                out_ref):
    iota_v = jax.lax.broadcasted_iota(jnp.int32, (1, VOCAB), 1)
    onehot_a = (atoms_ref[...] == iota_v).astype(_BF)
    iota_g = jax.lax.broadcasted_iota(jnp.int32, (G, 1), 0)
    onehot_bT = (batch_ref[...] == iota_g).astype(_BF)
    wcnt = jnp.dot(onehot_bT, onehot_a,
                   preferred_element_type=_F32)
    countf = jnp.sum(wcnt, axis=0, keepdims=True)

    hs = emb16T_ref[...]
    s = jnp.ones((1, VOCAB), _F32)
    for i in range(L):
        z = jax.nn.relu(_dot_bf(w1T_ref[i], hs) + b1_ref[i])
        t = _dot_bf(w2T_ref[i], z) + b2_ref[pl.ds(i, 1), :]
        m = jax.nn.sigmoid(t)
        mask = m > 0.5
        k = jnp.sum(jnp.where(mask, countf, 0.0))
        cond = k > 1.5
        maskf = mask.astype(_F32)
        h_master = hs * maskf
        expanded = jnp.where(mask, h_master, jnp.zeros_like(hs))
        blended = (1.0 - m) * hs + m * expanded
        hs = jnp.where(cond, blended, hs)
        lscale = jnp.where(mask, 1.0, 1.0 - m)
        s = jnp.where(cond, s * lscale, s)

    wgv = wcnt * s
    pooled = jnp.dot(wgv, emb_ref[...], precision=_HI,
                     preferred_element_type=_F32)
    hid = jax.nn.relu(_dot_bf(pooled, pw1_ref[...]) + pb1_ref[...])
    out_ref[...] = _dot_bf(hid, pw2_ref[...]) + pb2_ref[...]


def kernel(atoms, pos, edge_index, batch, emb, ms_W1, ms_b1, ms_W2, ms_b2,
           vg_Wq, vg_Wk, pred_W1, pred_b1, pred_W2, pred_b2):
    del pos, edge_index, vg_Wq, vg_Wk
    atoms_col = atoms.astype(jnp.int32).reshape(N, 1)
    batch_row = batch.astype(jnp.int32).reshape(1, N)
    emb16T = emb[:, :SDIM].T
    w1T = ms_W1.transpose(0, 2, 1)
    b1r = ms_b1.reshape(L, HID, 1)
    w2T = ms_W2.reshape(L, 1, HID)
    pb1r = pred_b1.reshape(1, EMB // 2)
    pb2r = pred_b2.reshape(1, 1)
    return pl.pallas_call(
        _hmp_kernel,
        out_shape=jax.ShapeDtypeStruct((G, 1), jnp.float32),
    )(atoms_col, batch_row, emb, emb16T, w1T, b1r, w2T, ms_b2, pred_W1,
      pb1r, pred_W2, pb2r)


# speedup vs baseline: 2.0102x; 2.0102x over previous
import jax
import jax.numpy as jnp
from jax.experimental import pallas as pl

N = 10000
EMB = 128
SDIM = 16
HID = 32
L = 5
VOCAB = 100
G = 64

_HI = jax.lax.Precision.HIGHEST
_BF = jnp.bfloat16
_F32 = jnp.float32


def _dot_bf(a, b):
    return jnp.dot(a.astype(_BF), b.astype(_BF), preferred_element_type=_F32)


def _hmp_kernel(atoms_ref, batch_ref, emb_ref, w1_ref, b1_ref, w2_ref,
                b2_ref, pw1_ref, pb1_ref, pw2_ref, pb2_ref, out_ref):
    iota_v = jax.lax.broadcasted_iota(jnp.int32, (VOCAB, 1), 0)
    onehot_aT = (atoms_ref[...] == iota_v).astype(_BF)
    iota_g = jax.lax.broadcasted_iota(jnp.int32, (G, 1), 0)
    onehot_bT = (batch_ref[...] == iota_g).astype(_BF)
    wcntT = jax.lax.dot_general(onehot_aT, onehot_bT,
                                (((1,), (1,)), ((), ())),
                                preferred_element_type=_F32)
    countf = jnp.sum(wcntT, axis=1, keepdims=True)

    hs = emb_ref[:, :SDIM]
    s = jnp.ones((VOCAB, 1), _F32)
    for i in range(L):
        z = jax.nn.relu(_dot_bf(hs, w1_ref[i])
                        + b1_ref[pl.ds(i, 1), :])
        t = _dot_bf(z, w2_ref[i]) + b2_ref[pl.ds(i, 1), :]
        m = jax.nn.sigmoid(t)
        mask = m > 0.5
        k = jnp.sum(jnp.where(mask, countf, 0.0))
        cond = k > 1.5
        maskf = mask.astype(_F32)
        h_master = hs * maskf
        expanded = jnp.where(mask, h_master, jnp.zeros_like(hs))
        blended = (1.0 - m) * hs + m * expanded
        hs = jnp.where(cond, blended, hs)
        lscale = jnp.where(mask, 1.0, 1.0 - m)
        s = jnp.where(cond, s * lscale, s)

    wgvT = wcntT * s
    pooled = jax.lax.dot_general(wgvT, emb_ref[...],
                                 (((0,), (0,)), ((), ())),
                                 precision=_HI,
                                 preferred_element_type=_F32)
    hid = jax.nn.relu(_dot_bf(pooled, pw1_ref[...]) + pb1_ref[...])
    out_ref[...] = _dot_bf(hid, pw2_ref[...]) + pb2_ref[...]


def kernel(atoms, pos, edge_index, batch, emb, ms_W1, ms_b1, ms_W2, ms_b2,
           vg_Wq, vg_Wk, pred_W1, pred_b1, pred_W2, pred_b2):
    del pos, edge_index, vg_Wq, vg_Wk
    atoms_row = atoms.astype(jnp.int32).reshape(1, N)
    batch_row = batch.astype(jnp.int32).reshape(1, N)
    pb1r = pred_b1.reshape(1, EMB // 2)
    pb2r = pred_b2.reshape(1, 1)
    return pl.pallas_call(
        _hmp_kernel,
        out_shape=jax.ShapeDtypeStruct((G, 1), jnp.float32),
    )(atoms_row, batch_row, emb, ms_W1, ms_b1, ms_W2, ms_b2, pred_W1,
      pb1r, pred_W2, pb2r)


# speedup vs baseline: 2.3159x; 1.1521x over previous
import jax
import jax.numpy as jnp
from jax.experimental import pallas as pl

N = 10000
EMB = 128
SDIM = 16
HID = 32
L = 5
VOCAB = 100
G = 64

_HI = jax.lax.Precision.HIGHEST
_BF = jnp.bfloat16
_F32 = jnp.float32


def _dot_bf(a, b):
    return jnp.dot(a.astype(_BF), b.astype(_BF), preferred_element_type=_F32)


def _hmp_kernel(atoms_ref, batch_ref, emb_ref, w1_ref, b1_ref, w2_ref,
                b2_ref, pw1_ref, pb1_ref, pw2_ref, pb2_ref, out_ref):
    atoms_row = atoms_ref[...].reshape(1, N)
    batch_row = batch_ref[...].reshape(1, N)
    iota_v = jax.lax.broadcasted_iota(jnp.int32, (VOCAB, 1), 0)
    onehot_aT = (atoms_row == iota_v).astype(_BF)
    iota_g = jax.lax.broadcasted_iota(jnp.int32, (G, 1), 0)
    onehot_bT = (batch_row == iota_g).astype(_BF)
    wcntT = jax.lax.dot_general(onehot_aT, onehot_bT,
                                (((1,), (1,)), ((), ())),
                                preferred_element_type=_F32)
    countf = jnp.sum(wcntT, axis=1, keepdims=True)

    hs = emb_ref[:, :SDIM]
    s = jnp.ones((VOCAB, 1), _F32)
    for i in range(L):
        z = jax.nn.relu(_dot_bf(hs, w1_ref[i])
                        + b1_ref[pl.ds(i, 1), :])
        t = _dot_bf(z, w2_ref[i]) + b2_ref[pl.ds(i, 1), :]
        m = jax.nn.sigmoid(t)
        mask = m > 0.5
        k = jnp.sum(jnp.where(mask, countf, 0.0))
        cond = k > 1.5
        maskf = mask.astype(_F32)
        h_master = hs * maskf
        expanded = jnp.where(mask, h_master, jnp.zeros_like(hs))
        blended = (1.0 - m) * hs + m * expanded
        hs = jnp.where(cond, blended, hs)
        lscale = jnp.where(mask, 1.0, 1.0 - m)
        s = jnp.where(cond, s * lscale, s)

    wgvT = wcntT * s
    pooled = jax.lax.dot_general(wgvT, emb_ref[...],
                                 (((0,), (0,)), ((), ())),
                                 precision=_HI,
                                 preferred_element_type=_F32)
    hid = jax.nn.relu(_dot_bf(pooled, pw1_ref[...])
                      + pb1_ref[...].reshape(1, EMB // 2))
    out_ref[...] = (_dot_bf(hid, pw2_ref[...])
                    + pb2_ref[...].reshape(1, 1))


def kernel(atoms, pos, edge_index, batch, emb, ms_W1, ms_b1, ms_W2, ms_b2,
           vg_Wq, vg_Wk, pred_W1, pred_b1, pred_W2, pred_b2):
    del pos, edge_index, vg_Wq, vg_Wk
    return pl.pallas_call(
        _hmp_kernel,
        out_shape=jax.ShapeDtypeStruct((G, 1), jnp.float32),
    )(atoms.astype(jnp.int32), batch.astype(jnp.int32), emb, ms_W1, ms_b1,
      ms_W2, ms_b2, pred_W1, pred_b1, pred_W2, pred_b2)
